# baseline (speedup 1.0000x reference)
import functools

import jax
import jax.numpy as jnp
from jax import lax
from jax.experimental import pallas as pl
from jax.experimental.pallas import tpu as pltpu
from jax.experimental.pallas import tpu_sc as plsc

NUM_CORES = 2
NUM_SUBCORES = 16
LANES = 16


@functools.lru_cache(maxsize=None)
def _build(B, L, D, V):
    N = B * L
    NW = NUM_CORES * NUM_SUBCORES
    rows_per_w = B // NW
    elems_per_w = rows_per_w * L
    NJ = D // LANES
    pieces = []
    off = 0
    while off < L:
        sz = min(128, L - off)
        pieces.append((off, sz))
        off += sz

    mesh = plsc.VectorSubcoreMesh(
        core_axis_name="c", subcore_axis_name="s",
        num_cores=NUM_CORES, num_subcores=NUM_SUBCORES)

    NBUF = 3

    def body(seq_hbm, seg_hbm, tok_hbm, segtab_hbm, pos_hbm, out_hbm,
             tokidx_v, segidx_v, rows_a, rows_b, rows_c, pos_v, segtab_v,
             gsem_a, gsem_b, gsem_c, wsem_a, wsem_b, wsem_c):
        wid = lax.axis_index("s") * NUM_CORES + lax.axis_index("c")
        base0 = wid * elems_per_w
        pltpu.sync_copy(seq_hbm.at[pl.ds(base0, elems_per_w)], tokidx_v)
        pltpu.sync_copy(seg_hbm.at[pl.ds(base0, elems_per_w)], segidx_v)
        pltpu.sync_copy(pos_hbm.at[pl.ds(0, L)], pos_v)
        pltpu.sync_copy(segtab_hbm, segtab_v)
        segrows = [[segtab_v[s, pl.ds(LANES * j, LANES)] for j in range(NJ)]
                   for s in range(3)]

        rows = [rows_a, rows_b, rows_c]
        gsem = [gsem_a, gsem_b, gsem_c]
        wsem = [wsem_a, wsem_b, wsem_c]
        pend_g = [None] * NBUF
        pend_w = [None] * NBUF

        def fire_gather(c, buf):
            pend_g[buf] = [
                pltpu.async_copy(
                    tok_hbm.at[tokidx_v.at[pl.ds(c * L + o, sz)]],
                    rows[buf].at[pl.ds(o, sz)], gsem[buf])
                for (o, sz) in pieces
            ]

        for c0 in range(NBUF - 1):
            fire_gather(c0, c0)
        for c in range(rows_per_w):
            buf = c % NBUF
            if c + NBUF - 1 < rows_per_w:
                nb = (c + NBUF - 1) % NBUF
                if pend_w[nb] is not None:
                    pend_w[nb].wait()
                fire_gather(c + NBUF - 1, nb)
            for cp in pend_g[buf]:
                cp.wait()

            def elem(i, carry, c=c, buf=buf):
                svec = plsc.load_gather(
                    segidx_v, [jnp.full((LANES,), c * L, jnp.int32) + i])
                m1 = svec == 1
                m2 = svec == 2
                for j in range(NJ):
                    t = jnp.where(m2, segrows[2][j],
                                  jnp.where(m1, segrows[1][j], segrows[0][j]))
                    t = t + pos_v[i, pl.ds(LANES * j, LANES)]
                    plsc.addupdate(rows[buf].at[i, pl.ds(LANES * j, LANES)], t)
                return carry

            lax.fori_loop(0, L, elem, 0)
            pend_w[buf] = pltpu.async_copy(
                rows[buf], out_hbm.at[pl.ds(base0 + c * L, L)], wsem[buf])
        for pw in pend_w:
            if pw is not None:
                pw.wait()

    return pl.kernel(
        body,
        out_type=jax.ShapeDtypeStruct((N, D), jnp.float32),
        mesh=mesh,
        compiler_params=pltpu.CompilerParams(needs_layout_passes=False),
        scratch_types=[
            pltpu.VMEM((elems_per_w,), jnp.int32),
            pltpu.VMEM((elems_per_w,), jnp.int32),
            pltpu.VMEM((L, D), jnp.float32),
            pltpu.VMEM((L, D), jnp.float32),
            pltpu.VMEM((L, D), jnp.float32),
            pltpu.VMEM((L, D), jnp.float32),
            pltpu.VMEM((3, D), jnp.float32),
            pltpu.SemaphoreType.DMA,
            pltpu.SemaphoreType.DMA,
            pltpu.SemaphoreType.DMA,
            pltpu.SemaphoreType.DMA,
            pltpu.SemaphoreType.DMA,
            pltpu.SemaphoreType.DMA,
        ],
    )


def kernel(seq, seg_label, token_table, seg_table, pos_emb):
    B, L = seq.shape
    V, D = token_table.shape
    seqf = seq.reshape(-1).astype(jnp.int32)
    segf = seg_label.reshape(-1).astype(jnp.int32)
    out = _build(B, L, D, V)(seqf, segf, token_table, seg_table, pos_emb)
    return out.reshape(B, L, D)


# speedup vs baseline: 1.0577x; 1.0577x over previous
import functools

import jax
import jax.numpy as jnp
from jax import lax
from jax.experimental import pallas as pl
from jax.experimental.pallas import tpu as pltpu
from jax.experimental.pallas import tpu_sc as plsc

NUM_CORES = 2
NUM_SUBCORES = 16
LANES = 16
NBUF = 4
AHEAD = 2


@functools.lru_cache(maxsize=None)
def _build(B, L, D, V):
    N = B * L
    NW = NUM_CORES * NUM_SUBCORES
    seqs_per_w = B // NW
    elems_per_w = seqs_per_w * L
    NJ = D // LANES
    half = [(0, 104), (104, L - 104)]
    CMAX = 104
    chunks = [(s * L + l0, l0, cl) for s in range(seqs_per_w)
              for (l0, cl) in half]
    NCH = len(chunks)

    mesh = plsc.VectorSubcoreMesh(
        core_axis_name="c", subcore_axis_name="s",
        num_cores=NUM_CORES, num_subcores=NUM_SUBCORES)

    def body(seq_hbm, seg_hbm, tok_hbm, segtab_hbm, pos_hbm, out_hbm,
             tokidx_v, segidx_v, rows_a, rows_b, rows_c, rows_d, pos_v,
             segtab_v, gsem_a, gsem_b, gsem_c, gsem_d,
             wsem_a, wsem_b, wsem_c, wsem_d):
        wid = lax.axis_index("s") * NUM_CORES + lax.axis_index("c")
        base0 = wid * elems_per_w
        pltpu.sync_copy(seq_hbm.at[pl.ds(base0, elems_per_w)], tokidx_v)
        pltpu.sync_copy(seg_hbm.at[pl.ds(base0, elems_per_w)], segidx_v)
        pltpu.sync_copy(pos_hbm.at[pl.ds(0, L)], pos_v)
        pltpu.sync_copy(segtab_hbm, segtab_v)
        segrows = [[segtab_v[s, pl.ds(LANES * j, LANES)] for j in range(NJ)]
                   for s in range(3)]

        rows = [rows_a, rows_b, rows_c, rows_d]
        gsem = [gsem_a, gsem_b, gsem_c, gsem_d]
        wsem = [wsem_a, wsem_b, wsem_c, wsem_d]
        pend_g = [None] * NBUF
        pend_w = [None] * NBUF

        def fire_gather(c, buf):
            eoff, _, cl = chunks[c]
            pend_g[buf] = pltpu.async_copy(
                tok_hbm.at[tokidx_v.at[pl.ds(eoff, cl)]],
                rows[buf].at[pl.ds(0, cl)], gsem[buf])

        for c0 in range(AHEAD):
            fire_gather(c0, c0)
        for c in range(NCH):
            buf = c % NBUF
            eoff, l0, cl = chunks[c]
            pend_g[buf].wait()

            def elem(i, carry, eoff=eoff, l0=l0, buf=buf):
                svec = plsc.load_gather(
                    segidx_v, [jnp.full((LANES,), eoff, jnp.int32) + i])
                m1 = svec == 1
                m2 = svec == 2
                for j in range(NJ):
                    t = jnp.where(m2, segrows[2][j],
                                  jnp.where(m1, segrows[1][j], segrows[0][j]))
                    t = t + pos_v[l0 + i, pl.ds(LANES * j, LANES)]
                    plsc.addupdate(rows[buf].at[i, pl.ds(LANES * j, LANES)], t)
                return carry

            lax.fori_loop(0, cl, elem, 0, unroll=2)
            if c + AHEAD < NCH:
                nb = (c + AHEAD) % NBUF
                if pend_w[nb] is not None:
                    pend_w[nb].wait()
                fire_gather(c + AHEAD, nb)
            pend_w[buf] = pltpu.async_copy(
                rows[buf].at[pl.ds(0, cl)],
                out_hbm.at[pl.ds(base0 + eoff, cl)], wsem[buf])
        for pw in pend_w:
            if pw is not None:
                pw.wait()

    return pl.kernel(
        body,
        out_type=jax.ShapeDtypeStruct((N, D), jnp.float32),
        mesh=mesh,
        compiler_params=pltpu.CompilerParams(needs_layout_passes=False),
        scratch_types=[
            pltpu.VMEM((elems_per_w,), jnp.int32),
            pltpu.VMEM((elems_per_w,), jnp.int32),
            pltpu.VMEM((CMAX, D), jnp.float32),
            pltpu.VMEM((CMAX, D), jnp.float32),
            pltpu.VMEM((CMAX, D), jnp.float32),
            pltpu.VMEM((CMAX, D), jnp.float32),
            pltpu.VMEM((L, D), jnp.float32),
            pltpu.VMEM((3, D), jnp.float32),
            pltpu.SemaphoreType.DMA,
            pltpu.SemaphoreType.DMA,
            pltpu.SemaphoreType.DMA,
            pltpu.SemaphoreType.DMA,
            pltpu.SemaphoreType.DMA,
            pltpu.SemaphoreType.DMA,
            pltpu.SemaphoreType.DMA,
            pltpu.SemaphoreType.DMA,
        ],
    )


def kernel(seq, seg_label, token_table, seg_table, pos_emb):
    B, L = seq.shape
    V, D = token_table.shape
    seqf = seq.reshape(-1).astype(jnp.int32)
    segf = seg_label.reshape(-1).astype(jnp.int32)
    out = _build(B, L, D, V)(seqf, segf, token_table, seg_table, pos_emb)
    return out.reshape(B, L, D)


# speedup vs baseline: 1.2156x; 1.1493x over previous
import functools

import jax
import jax.numpy as jnp
from jax import lax
from jax.experimental import pallas as pl
from jax.experimental.pallas import tpu as pltpu
from jax.experimental.pallas import tpu_sc as plsc

NUM_CORES = 2
NUM_SUBCORES = 16
LANES = 16


@functools.lru_cache(maxsize=None)
def _build(B, L, D, V):
    N = B * L
    NW = NUM_CORES * NUM_SUBCORES
    rows_per_w = B // NW
    elems_per_w = rows_per_w * L
    NJ = D // LANES
    pieces = []
    off = 0
    while off < L:
        sz = min(128, L - off)
        pieces.append((off, sz))
        off += sz

    mesh = plsc.VectorSubcoreMesh(
        core_axis_name="c", subcore_axis_name="s",
        num_cores=NUM_CORES, num_subcores=NUM_SUBCORES)

    NBUF = 3

    def body(seq_hbm, seg_hbm, tok_hbm, segtab_hbm, pos_hbm, out_hbm,
             tokidx_v, segidx_v, rows_a, rows_b, rows_c, pos_v, segtab_v,
             gsem_a, gsem_b, gsem_c, wsem_a, wsem_b, wsem_c):
        wid = lax.axis_index("s") * NUM_CORES + lax.axis_index("c")
        base0 = wid * elems_per_w
        pltpu.sync_copy(seq_hbm.at[pl.ds(base0, elems_per_w)], tokidx_v)
        pltpu.sync_copy(seg_hbm.at[pl.ds(base0, elems_per_w)], segidx_v)
        pltpu.sync_copy(pos_hbm.at[pl.ds(0, L)], pos_v)
        pltpu.sync_copy(segtab_hbm, segtab_v)
        segrows = [[segtab_v[s, pl.ds(LANES * j, LANES)] for j in range(NJ)]
                   for s in range(3)]

        rows = [rows_a, rows_b, rows_c]
        gsem = [gsem_a, gsem_b, gsem_c]
        wsem = [wsem_a, wsem_b, wsem_c]
        pend_g = [None] * NBUF
        pend_w = [None] * NBUF

        def fire_gather(c, buf):
            pend_g[buf] = [
                pltpu.async_copy(
                    tok_hbm.at[tokidx_v.at[pl.ds(c * L + o, sz)]],
                    rows[buf].at[pl.ds(o, sz)], gsem[buf])
                for (o, sz) in pieces
            ]

        for c0 in range(NBUF - 1):
            fire_gather(c0, c0)
        for c in range(rows_per_w):
            buf = c % NBUF
            for cp in pend_g[buf]:
                cp.wait()

            def elem(i, carry, c=c, buf=buf):
                svec = plsc.load_gather(
                    segidx_v, [jnp.full((LANES,), c * L, jnp.int32) + i])
                m1 = svec == 1
                m2 = svec == 2
                for j in range(NJ):
                    t = jnp.where(m2, segrows[2][j],
                                  jnp.where(m1, segrows[1][j], segrows[0][j]))
                    t = t + pos_v[i, pl.ds(LANES * j, LANES)]
                    plsc.addupdate(rows[buf].at[i, pl.ds(LANES * j, LANES)], t)
                return carry

            lax.fori_loop(0, L, elem, 0)
            if c + NBUF - 1 < rows_per_w:
                nb = (c + NBUF - 1) % NBUF
                if pend_w[nb] is not None:
                    pend_w[nb].wait()
                fire_gather(c + NBUF - 1, nb)
            pend_w[buf] = pltpu.async_copy(
                rows[buf], out_hbm.at[pl.ds(base0 + c * L, L)], wsem[buf])
        for pw in pend_w:
            if pw is not None:
                pw.wait()

    return pl.kernel(
        body,
        out_type=jax.ShapeDtypeStruct((N, D), jnp.float32),
        mesh=mesh,
        compiler_params=pltpu.CompilerParams(needs_layout_passes=False),
        scratch_types=[
            pltpu.VMEM((elems_per_w,), jnp.int32),
            pltpu.VMEM((elems_per_w,), jnp.int32),
            pltpu.VMEM((L, D), jnp.float32),
            pltpu.VMEM((L, D), jnp.float32),
            pltpu.VMEM((L, D), jnp.float32),
            pltpu.VMEM((L, D), jnp.float32),
            pltpu.VMEM((3, D), jnp.float32),
            pltpu.SemaphoreType.DMA,
            pltpu.SemaphoreType.DMA,
            pltpu.SemaphoreType.DMA,
            pltpu.SemaphoreType.DMA,
            pltpu.SemaphoreType.DMA,
            pltpu.SemaphoreType.DMA,
        ],
    )


def kernel(seq, seg_label, token_table, seg_table, pos_emb):
    B, L = seq.shape
    V, D = token_table.shape
    seqf = seq.reshape(-1).astype(jnp.int32)
    segf = seg_label.reshape(-1).astype(jnp.int32)
    out = _build(B, L, D, V)(seqf, segf, token_table, seg_table, pos_emb)
    return out.reshape(B, L, D)
